# baseline (speedup 1.0000x reference)
import functools

import jax
import jax.numpy as jnp
from jax import lax
from jax.experimental import pallas as pl
from jax.experimental.pallas import tpu as pltpu

_K = 20
_ROW_BLOCK = 512


def _body(er_ref, et_ref, ap_ref, al_ref, out_ref):
    er = er_ref[0]
    et = et_ref[0]
    scores = lax.dot_general(
        er, et, (((1,), (0,)), ((), ())),
        preferred_element_type=jnp.float32)
    a = jnp.maximum(scores, 0.0)
    bits = lax.bitcast_convert_type(a, jnp.int32)

    t = jnp.zeros((a.shape[0], 1), jnp.int32)
    for bit in range(30, -1, -1):
        cand = t | jnp.int32(1 << bit)
        cnt = jnp.sum((bits >= cand).astype(jnp.int32), axis=1, keepdims=True)
        t = jnp.where(cnt >= _K, cand, t)

    gt = bits > t
    cnt_gt = jnp.sum(gt.astype(jnp.int32), axis=1, keepdims=True)
    rem = _K - cnt_gt
    eq = bits == t
    cnt_eq = jnp.sum(eq.astype(jnp.int32), axis=1, keepdims=True)

    def _ties_slow(eq, rem):
        col = lax.broadcasted_iota(jnp.int32, eq.shape, 1)
        c = jnp.zeros((eq.shape[0], 1), jnp.int32)
        for bit in range(10, -1, -1):
            cand = c | jnp.int32(1 << bit)
            cnt = jnp.sum((eq & (col <= cand)).astype(jnp.int32),
                          axis=1, keepdims=True)
            c = jnp.where(cnt <= rem, cand, c)
        return eq & (col <= c)

    any_dup = jnp.any(cnt_gt + cnt_eq != _K)

    m = jnp.max(a, axis=1, keepdims=True)
    ap = ap_ref[...]
    rs = jnp.sum(ap, axis=1, keepdims=True) + 1e-8
    al = jnp.full((1, 1), al_ref[0])
    ac = 1.0 / (1.0 + jnp.exp(-al))
    phys_scale = ac / rs

    def _emit(ties):
        sel = gt | ties
        e = jnp.where(sel, jnp.exp(a - m), 0.0)
        s = jnp.sum(e, axis=1, keepdims=True)
        dyn_scale = (1.0 - ac) / s
        out_ref[0] = ap * phys_scale + e * dyn_scale

    @pl.when(jnp.logical_not(any_dup))
    def _():
        _emit(eq)

    @pl.when(any_dup)
    def _():
        _emit(_ties_slow(eq, rem))


def _build(B, T, N, interpret=False):
    R = _ROW_BLOCK
    nb = N // R
    grid = (nb, B)
    return pl.pallas_call(
        _body,
        grid=grid,
        in_specs=[
            pl.BlockSpec((1, R, 16), lambda i, bb: (bb, i, 0)),
            pl.BlockSpec((1, 16, N), lambda i, bb: (bb, 0, 0)),
            pl.BlockSpec((R, N), lambda i, bb: (i, 0)),
            pl.BlockSpec(memory_space=pltpu.SMEM),
        ],
        out_specs=pl.BlockSpec((1, R, N), lambda i, bb: (bb, i, 0)),
        out_shape=jax.ShapeDtypeStruct((B, N, N), jnp.float32),
        compiler_params=pltpu.CompilerParams(
            dimension_semantics=("arbitrary", "arbitrary")),
        interpret=interpret,
    )


@jax.jit
def kernel(x, A_physical, W, b, alpha):
    B, T, N, _ = x.shape
    state = x.mean(axis=1)
    emb = jnp.tanh(state @ W.T + b)
    embT = emb.transpose(0, 2, 1)
    al = alpha.reshape(1)
    return _build(B, T, N)(emb, embT, A_physical, al)


# speedup vs baseline: 1.0082x; 1.0082x over previous
import functools

import jax
import jax.numpy as jnp
from jax import lax
from jax.experimental import pallas as pl
from jax.experimental.pallas import tpu as pltpu

_K = 20
_ROW_BLOCK = 128


def _body(er_ref, et_ref, ap_ref, al_ref, out_ref):
    er = er_ref[0]
    et = et_ref[0]
    scores = lax.dot_general(
        er, et, (((1,), (0,)), ((), ())),
        preferred_element_type=jnp.float32)
    a = jnp.maximum(scores, 0.0)
    bits = lax.bitcast_convert_type(a, jnp.int32)

    t = jnp.zeros((a.shape[0], 1), jnp.int32)
    for bit in range(30, -1, -1):
        cand = t | jnp.int32(1 << bit)
        cnt = jnp.sum((bits >= cand).astype(jnp.int32), axis=1, keepdims=True)
        t = jnp.where(cnt >= _K, cand, t)

    gt = bits > t
    cnt_gt = jnp.sum(gt.astype(jnp.int32), axis=1, keepdims=True)
    rem = _K - cnt_gt
    eq = bits == t
    cnt_eq = jnp.sum(eq.astype(jnp.int32), axis=1, keepdims=True)

    def _ties_slow(eq, rem):
        col = lax.broadcasted_iota(jnp.int32, eq.shape, 1)
        c = jnp.zeros((eq.shape[0], 1), jnp.int32)
        for bit in range(10, -1, -1):
            cand = c | jnp.int32(1 << bit)
            cnt = jnp.sum((eq & (col <= cand)).astype(jnp.int32),
                          axis=1, keepdims=True)
            c = jnp.where(cnt <= rem, cand, c)
        return eq & (col <= c)

    any_dup = jnp.any(cnt_gt + cnt_eq != _K)

    m = jnp.max(a, axis=1, keepdims=True)
    ap = ap_ref[...]
    rs = jnp.sum(ap, axis=1, keepdims=True) + 1e-8
    al = jnp.full((1, 1), al_ref[0])
    ac = 1.0 / (1.0 + jnp.exp(-al))
    phys_scale = ac / rs

    def _emit(ties):
        sel = gt | ties
        e = jnp.where(sel, jnp.exp(a - m), 0.0)
        s = jnp.sum(e, axis=1, keepdims=True)
        dyn_scale = (1.0 - ac) / s
        out_ref[0] = ap * phys_scale + e * dyn_scale

    @pl.when(jnp.logical_not(any_dup))
    def _():
        _emit(eq)

    @pl.when(any_dup)
    def _():
        _emit(_ties_slow(eq, rem))


def _build(B, T, N, interpret=False):
    R = _ROW_BLOCK
    nb = N // R
    grid = (nb, B)
    return pl.pallas_call(
        _body,
        grid=grid,
        in_specs=[
            pl.BlockSpec((1, R, 16), lambda i, bb: (bb, i, 0)),
            pl.BlockSpec((1, 16, N), lambda i, bb: (bb, 0, 0)),
            pl.BlockSpec((R, N), lambda i, bb: (i, 0)),
            pl.BlockSpec(memory_space=pltpu.SMEM),
        ],
        out_specs=pl.BlockSpec((1, R, N), lambda i, bb: (bb, i, 0)),
        out_shape=jax.ShapeDtypeStruct((B, N, N), jnp.float32),
        compiler_params=pltpu.CompilerParams(
            dimension_semantics=("arbitrary", "arbitrary")),
        interpret=interpret,
    )


@jax.jit
def kernel(x, A_physical, W, b, alpha):
    B, T, N, _ = x.shape
    state = x.mean(axis=1)
    emb = jnp.tanh(state @ W.T + b)
    embT = emb.transpose(0, 2, 1)
    al = alpha.reshape(1)
    return _build(B, T, N)(emb, embT, A_physical, al)


# speedup vs baseline: 1.0198x; 1.0115x over previous
import functools

import jax
import jax.numpy as jnp
from jax import lax
from jax.experimental import pallas as pl
from jax.experimental.pallas import tpu as pltpu

_K = 20
_ROW_BLOCK = 256


def _body(er_ref, et_ref, ap_ref, al_ref, out_ref):
    er = er_ref[0]
    et = et_ref[0]
    scores = lax.dot_general(
        er, et, (((1,), (0,)), ((), ())),
        preferred_element_type=jnp.float32)
    a = jnp.maximum(scores, 0.0)
    bits = lax.bitcast_convert_type(a, jnp.int32)

    t = jnp.zeros((a.shape[0], 1), jnp.int32)
    for bit in range(30, -1, -1):
        cand = t | jnp.int32(1 << bit)
        cnt = jnp.sum((bits >= cand).astype(jnp.int32), axis=1, keepdims=True)
        t = jnp.where(cnt >= _K, cand, t)

    gt = bits > t
    cnt_gt = jnp.sum(gt.astype(jnp.int32), axis=1, keepdims=True)
    rem = _K - cnt_gt
    eq = bits == t
    cnt_eq = jnp.sum(eq.astype(jnp.int32), axis=1, keepdims=True)

    def _ties_slow(eq, rem):
        col = lax.broadcasted_iota(jnp.int32, eq.shape, 1)
        c = jnp.zeros((eq.shape[0], 1), jnp.int32)
        for bit in range(10, -1, -1):
            cand = c | jnp.int32(1 << bit)
            cnt = jnp.sum((eq & (col <= cand)).astype(jnp.int32),
                          axis=1, keepdims=True)
            c = jnp.where(cnt <= rem, cand, c)
        return eq & (col <= c)

    any_dup = jnp.any(cnt_gt + cnt_eq != _K)

    m = jnp.max(a, axis=1, keepdims=True)
    ap = ap_ref[...]
    rs = jnp.sum(ap, axis=1, keepdims=True) + 1e-8
    al = jnp.full((1, 1), al_ref[0])
    ac = 1.0 / (1.0 + jnp.exp(-al))
    phys_scale = ac / rs

    def _emit(ties):
        sel = gt | ties
        e = jnp.where(sel, jnp.exp(a - m), 0.0)
        s = jnp.sum(e, axis=1, keepdims=True)
        dyn_scale = (1.0 - ac) / s
        out_ref[0] = ap * phys_scale + e * dyn_scale

    @pl.when(jnp.logical_not(any_dup))
    def _():
        _emit(eq)

    @pl.when(any_dup)
    def _():
        _emit(_ties_slow(eq, rem))


def _build(B, T, N, interpret=False):
    R = _ROW_BLOCK
    nb = N // R
    grid = (nb, B)
    return pl.pallas_call(
        _body,
        grid=grid,
        in_specs=[
            pl.BlockSpec((1, R, 16), lambda i, bb: (bb, i, 0)),
            pl.BlockSpec((1, 16, N), lambda i, bb: (bb, 0, 0)),
            pl.BlockSpec((R, N), lambda i, bb: (i, 0)),
            pl.BlockSpec(memory_space=pltpu.SMEM),
        ],
        out_specs=pl.BlockSpec((1, R, N), lambda i, bb: (bb, i, 0)),
        out_shape=jax.ShapeDtypeStruct((B, N, N), jnp.float32),
        compiler_params=pltpu.CompilerParams(
            dimension_semantics=("arbitrary", "arbitrary")),
        interpret=interpret,
    )


@jax.jit
def kernel(x, A_physical, W, b, alpha):
    B, T, N, _ = x.shape
    state = x.mean(axis=1)
    emb = jnp.tanh(state @ W.T + b)
    embT = emb.transpose(0, 2, 1)
    al = alpha.reshape(1)
    return _build(B, T, N)(emb, embT, A_physical, al)


# speedup vs baseline: 1.0524x; 1.0319x over previous
import jax
import jax.numpy as jnp
from jax import lax
from jax.experimental import pallas as pl
from jax.experimental.pallas import tpu as pltpu

_K = 20
_ROW_BLOCK = 256


def _body(er_ref, et_ref, ap_ref, al_ref, out_ref):
    er = er_ref[0]
    et = et_ref[0]
    scores = lax.dot_general(
        er, et, (((1,), (0,)), ((), ())),
        preferred_element_type=jnp.float32)
    a = jnp.maximum(scores, 0.0)
    bits = lax.bitcast_convert_type(a, jnp.int32)

    t = jnp.zeros((a.shape[0], 1), jnp.int32)
    for bit in range(30, -1, -1):
        cand = t | jnp.int32(1 << bit)
        cnt = jnp.sum((bits >= cand).astype(jnp.int32), axis=1, keepdims=True)
        t = jnp.where(cnt >= _K, cand, t)

    ge = bits >= t
    cnt_ge = jnp.sum(ge.astype(jnp.int32), axis=1, keepdims=True)

    def _ties_slow():
        gt = bits > t
        cnt_gt = jnp.sum(gt.astype(jnp.int32), axis=1, keepdims=True)
        rem = _K - cnt_gt
        col = lax.broadcasted_iota(jnp.int32, ge.shape, 1)
        mcol = jnp.where(ge & jnp.logical_not(gt), col, jnp.int32(4095))
        c = jnp.zeros((ge.shape[0], 1), jnp.int32)
        for bit in range(10, -1, -1):
            cand = c | jnp.int32(1 << bit)
            cnt = jnp.sum((mcol <= cand).astype(jnp.int32),
                          axis=1, keepdims=True)
            c = jnp.where(cnt <= rem, cand, c)
        return gt | (mcol <= c)

    any_dup = jnp.any(cnt_ge != _K)

    m = jnp.max(a, axis=1, keepdims=True)
    ap = ap_ref[...]
    rs = jnp.sum(ap, axis=1, keepdims=True) + 1e-8
    al = jnp.full((1, 1), al_ref[0])
    ac = 1.0 / (1.0 + jnp.exp(-al))
    phys_scale = ac / rs

    def _emit(sel):
        e = jnp.where(sel, jnp.exp(a - m), 0.0)
        s = jnp.sum(e, axis=1, keepdims=True)
        dyn_scale = (1.0 - ac) / s
        out_ref[0] = ap * phys_scale + e * dyn_scale

    @pl.when(jnp.logical_not(any_dup))
    def _():
        _emit(ge)

    @pl.when(any_dup)
    def _():
        _emit(_ties_slow())


def _build(B, T, N, interpret=False):
    R = _ROW_BLOCK
    nb = N // R
    grid = (nb, B)
    return pl.pallas_call(
        _body,
        grid=grid,
        in_specs=[
            pl.BlockSpec((1, R, 16), lambda i, bb: (bb, i, 0)),
            pl.BlockSpec((1, 16, N), lambda i, bb: (bb, 0, 0)),
            pl.BlockSpec((R, N), lambda i, bb: (i, 0)),
            pl.BlockSpec(memory_space=pltpu.SMEM),
        ],
        out_specs=pl.BlockSpec((1, R, N), lambda i, bb: (bb, i, 0)),
        out_shape=jax.ShapeDtypeStruct((B, N, N), jnp.float32),
        compiler_params=pltpu.CompilerParams(
            dimension_semantics=("arbitrary", "arbitrary")),
        interpret=interpret,
    )


@jax.jit
def kernel(x, A_physical, W, b, alpha):
    B, T, N, _ = x.shape
    state = x.mean(axis=1)
    emb = jnp.tanh(state @ W.T + b)
    embT = emb.transpose(0, 2, 1)
    al = alpha.reshape(1)
    return _build(B, T, N)(emb, embT, A_physical, al)
